# baseline (speedup 1.0000x reference)
import jax
import jax.numpy as jnp
from jax.experimental import pallas as pl
from jax.experimental.pallas import tpu as pltpu


def _dma_copy_kernel(x_ref, o_ref, sem):
    pltpu.make_async_copy(x_ref, o_ref, sem).start()
    pltpu.make_async_copy(x_ref, o_ref, sem).wait()


def kernel(inputs_embeds, embed_table):
    del embed_table
    return pl.pallas_call(
        _dma_copy_kernel,
        in_specs=[pl.BlockSpec(memory_space=pl.ANY)],
        out_specs=pl.BlockSpec(memory_space=pl.ANY),
        scratch_shapes=[pltpu.SemaphoreType.DMA],
        out_shape=jax.ShapeDtypeStruct(inputs_embeds.shape, inputs_embeds.dtype),
    )(inputs_embeds)
